# Initial kernel scaffold; baseline (speedup 1.0000x reference)
import functools
import jax
import jax.numpy as jnp
from jax.experimental import pallas as pl
from jax.experimental.pallas import tpu as pltpu

HID = 128
HEADS = 4
CH = 32
LAYERS = 4


def _mm_bias_body(x_ref, w_ref, b_ref, o_ref):
    o_ref[...] = jnp.dot(x_ref[...], w_ref[...],
                         preferred_element_type=jnp.float32) + b_ref[...]


def _mm_bias(x, w, b, block_rows):
    R, K = x.shape
    M = w.shape[1]
    grid = (pl.cdiv(R, block_rows),)
    return pl.pallas_call(
        _mm_bias_body,
        grid=grid,
        in_specs=[
            pl.BlockSpec((block_rows, K), lambda i: (i, 0)),
            pl.BlockSpec((K, M), lambda i: (0, 0)),
            pl.BlockSpec((1, M), lambda i: (0, 0)),
        ],
        out_specs=pl.BlockSpec((block_rows, M), lambda i: (i, 0)),
        out_shape=jax.ShapeDtypeStruct((R, M), jnp.float32),
    )(x, w, b[None, :])


def _mm2_body(h_ref, wl_ref, bl_ref, wr_ref, br_ref, xl_ref, xr_ref):
    h = h_ref[...]
    xl_ref[...] = jnp.dot(h, wl_ref[...],
                          preferred_element_type=jnp.float32) + bl_ref[...]
    xr_ref[...] = jnp.dot(h, wr_ref[...],
                          preferred_element_type=jnp.float32) + br_ref[...]


def _mm2(h, wl, bl, wr, br, block_rows):
    R = h.shape[0]
    grid = (pl.cdiv(R, block_rows),)
    return pl.pallas_call(
        _mm2_body,
        grid=grid,
        in_specs=[
            pl.BlockSpec((block_rows, HID), lambda i: (i, 0)),
            pl.BlockSpec((HID, HID), lambda i: (0, 0)),
            pl.BlockSpec((1, HID), lambda i: (0, 0)),
            pl.BlockSpec((HID, HID), lambda i: (0, 0)),
            pl.BlockSpec((1, HID), lambda i: (0, 0)),
        ],
        out_specs=[
            pl.BlockSpec((block_rows, HID), lambda i: (i, 0)),
            pl.BlockSpec((block_rows, HID), lambda i: (i, 0)),
        ],
        out_shape=[
            jax.ShapeDtypeStruct((R, HID), jnp.float32),
            jax.ShapeDtypeStruct((R, HID), jnp.float32),
        ],
    )(h, wl, bl[None, :], wr, br[None, :])


def _alpha_body(g_ref, ea_ref, wf_ref, bf_ref, att_ref, ex_ref):
    s = g_ref[...] + jnp.dot(ea_ref[...], wf_ref[...],
                             preferred_element_type=jnp.float32) + bf_ref[...]
    m = jnp.where(s > 0, s, 0.2 * s)
    alpha = jnp.dot(m, att_ref[...], preferred_element_type=jnp.float32)
    ex_ref[...] = jnp.exp(alpha)


def _alpha(g, ea_pad, wf, bf, att_bd, block_rows):
    E = g.shape[0]
    KP = ea_pad.shape[1]
    grid = (pl.cdiv(E, block_rows),)
    return pl.pallas_call(
        _alpha_body,
        grid=grid,
        in_specs=[
            pl.BlockSpec((block_rows, HID), lambda i: (i, 0)),
            pl.BlockSpec((block_rows, KP), lambda i: (i, 0)),
            pl.BlockSpec((KP, HID), lambda i: (0, 0)),
            pl.BlockSpec((1, HID), lambda i: (0, 0)),
            pl.BlockSpec((HID, HEADS), lambda i: (0, 0)),
        ],
        out_specs=pl.BlockSpec((block_rows, HEADS), lambda i: (i, 0)),
        out_shape=jax.ShapeDtypeStruct((E, HEADS), jnp.float32),
    )(g, ea_pad, wf, bf[None, :], att_bd)


def _ln_body(h_ref, c_ref, bias_ref, g_ref, b_ref, o_ref):
    v = h_ref[...] + c_ref[...] + bias_ref[...]
    mu = jnp.mean(v, -1, keepdims=True)
    var = jnp.mean((v - mu) ** 2, -1, keepdims=True)
    o_ref[...] = (v - mu) / jnp.sqrt(var + 1e-5) * g_ref[...] + b_ref[...]


def _ln_res(h, c, bias, g, b, block_rows):
    R = h.shape[0]
    grid = (pl.cdiv(R, block_rows),)
    return pl.pallas_call(
        _ln_body,
        grid=grid,
        in_specs=[
            pl.BlockSpec((block_rows, HID), lambda i: (i, 0)),
            pl.BlockSpec((block_rows, HID), lambda i: (i, 0)),
            pl.BlockSpec((1, HID), lambda i: (0, 0)),
            pl.BlockSpec((1, HID), lambda i: (0, 0)),
            pl.BlockSpec((1, HID), lambda i: (0, 0)),
        ],
        out_specs=pl.BlockSpec((block_rows, HID), lambda i: (i, 0)),
        out_shape=jax.ShapeDtypeStruct((R, HID), jnp.float32),
    )(h, c, bias[None, :], g[None, :], b[None, :])


def _silu(v):
    return v * jax.nn.sigmoid(v)


def _final_body(s_ref, cnt_ref, wm1_ref, bm1_ref, wm2_ref, bm2_ref,
                w1_ref, b1_ref, w2_ref, b2_ref, w3_ref, b3_ref, o_ref):
    s = s_ref[...]
    cnt = jnp.maximum(cnt_ref[...], 1.0)
    mean = s / cnt
    gcat = jnp.concatenate([mean, s], axis=-1)
    z = _silu(jnp.dot(gcat, wm1_ref[...], preferred_element_type=jnp.float32)
              + bm1_ref[...])
    z = jnp.dot(z, wm2_ref[...], preferred_element_type=jnp.float32) + bm2_ref[...]
    outs = []
    for t in range(w1_ref.shape[0]):
        u = _silu(jnp.dot(z, w1_ref[t], preferred_element_type=jnp.float32)
                  + b1_ref[t][None, :])
        u = _silu(jnp.dot(u, w2_ref[t], preferred_element_type=jnp.float32)
                  + b2_ref[t][None, :])
        o = jnp.dot(u, w3_ref[t][:, None],
                    preferred_element_type=jnp.float32) + b3_ref[t][None, :]
        outs.append(o)
    o_ref[...] = jnp.concatenate(outs, axis=-1)


def _final(s, cnt, p, tasks):
    G = s.shape[0]
    w1 = jnp.stack([hd["W1"] for hd in p["heads"]])
    b1 = jnp.stack([hd["b1"] for hd in p["heads"]])
    w2 = jnp.stack([hd["W2"] for hd in p["heads"]])
    b2 = jnp.stack([hd["b2"] for hd in p["heads"]])
    w3 = jnp.stack([hd["W3"][:, 0] for hd in p["heads"]])
    b3 = jnp.stack([hd["b3"] for hd in p["heads"]])

    def full(*shape):
        return pl.BlockSpec(shape, lambda: tuple(0 for _ in shape))

    return pl.pallas_call(
        _final_body,
        in_specs=[
            full(G, HID), full(G, 1),
            full(2 * HID, HID), full(1, HID),
            full(HID, HID), full(1, HID),
            full(tasks, HID, HID), full(tasks, HID),
            full(tasks, HID, 64), full(tasks, 64),
            full(tasks, 64), full(tasks, 1),
        ],
        out_specs=full(G, tasks),
        out_shape=jax.ShapeDtypeStruct((G, tasks), jnp.float32),
    )(s, cnt[:, None], p["Wm1"], p["bm1"][None, :], p["Wm2"], p["bm2"][None, :],
      w1, b1, w2, b2, w3, b3)


def kernel(x, edge_index, edge_attr, batch, params):
    N = x.shape[0]
    E = edge_attr.shape[0]
    G = 1024
    TASKS = len(params["heads"])
    src = edge_index[0]
    dst = edge_index[1]

    xp = jnp.pad(x, ((0, 0), (0, 128 - x.shape[1])))
    w_ne = jnp.pad(params["W_ne"], ((0, 128 - x.shape[1]), (0, 0)))
    ea = jnp.pad(edge_attr, ((0, 0), (0, 16 - edge_attr.shape[1])))

    att = params["att"]
    att_bd = jnp.zeros((HID, HEADS), jnp.float32)
    for hh in range(HEADS):
        att_bd = att_bd.at[hh * CH:(hh + 1) * CH, hh].set(att[hh])

    h = _mm_bias(xp, w_ne, params["b_ne"], 1000)

    for li in range(LAYERS):
        c = params["convs"][li]
        wf = jnp.pad(params["W_ee"] @ c["We"], ((0, 6), (0, 0)))
        bf = params["b_ee"] @ c["We"]
        xl, xr = _mm2(h, c["Wl"], c["bl"], c["Wr"], c["br"], 1000)

        g = xl[src] + xr[dst]
        ex = _alpha(g, ea, wf, bf, att_bd, 3200)
        den = jax.ops.segment_sum(ex, dst, num_segments=N)
        a = ex / (den[dst] + 1e-16)
        msg = (xl[src].reshape(E, HEADS, CH) * a[..., None]).reshape(E, HID)
        num = jax.ops.segment_sum(msg, dst, num_segments=N)

        h = _ln_res(h, num, c["bias"], params["norms"][li]["g"],
                    params["norms"][li]["b"], 1000)

    s = jax.ops.segment_sum(h, batch, num_segments=G)
    cnt = jax.ops.segment_sum(jnp.ones((N,), jnp.float32), batch, num_segments=G)
    return _final(s, cnt, params, TASKS)


# speedup vs baseline: 1.9454x; 1.9454x over previous
import functools
import jax
import jax.numpy as jnp
from jax.experimental import pallas as pl
from jax.experimental.pallas import tpu as pltpu

HID = 128
HEADS = 4
CH = 32
LAYERS = 4


def _mm_bias_body(x_ref, w_ref, b_ref, o_ref):
    o_ref[...] = jnp.dot(x_ref[...], w_ref[...],
                         preferred_element_type=jnp.float32) + b_ref[...]


def _mm_bias(x, w, b, block_rows):
    R, K = x.shape
    M = w.shape[1]
    grid = (pl.cdiv(R, block_rows),)
    return pl.pallas_call(
        _mm_bias_body,
        grid=grid,
        in_specs=[
            pl.BlockSpec((block_rows, K), lambda i: (i, 0)),
            pl.BlockSpec((K, M), lambda i: (0, 0)),
            pl.BlockSpec((1, M), lambda i: (0, 0)),
        ],
        out_specs=pl.BlockSpec((block_rows, M), lambda i: (i, 0)),
        out_shape=jax.ShapeDtypeStruct((R, M), jnp.float32),
    )(x, w, b[None, :])


def _mm2_body(h_ref, wl_ref, bl_ref, wr_ref, br_ref, xl_ref, xr_ref):
    h = h_ref[...]
    xl_ref[...] = jnp.dot(h, wl_ref[...],
                          preferred_element_type=jnp.float32) + bl_ref[...]
    xr_ref[...] = jnp.dot(h, wr_ref[...],
                          preferred_element_type=jnp.float32) + br_ref[...]


def _mm2(h, wl, bl, wr, br, block_rows):
    R = h.shape[0]
    grid = (pl.cdiv(R, block_rows),)
    return pl.pallas_call(
        _mm2_body,
        grid=grid,
        in_specs=[
            pl.BlockSpec((block_rows, HID), lambda i: (i, 0)),
            pl.BlockSpec((HID, HID), lambda i: (0, 0)),
            pl.BlockSpec((1, HID), lambda i: (0, 0)),
            pl.BlockSpec((HID, HID), lambda i: (0, 0)),
            pl.BlockSpec((1, HID), lambda i: (0, 0)),
        ],
        out_specs=[
            pl.BlockSpec((block_rows, HID), lambda i: (i, 0)),
            pl.BlockSpec((block_rows, HID), lambda i: (i, 0)),
        ],
        out_shape=[
            jax.ShapeDtypeStruct((R, HID), jnp.float32),
            jax.ShapeDtypeStruct((R, HID), jnp.float32),
        ],
    )(h, wl, bl[None, :], wr, br[None, :])


def _alpha_body(g_ref, ea_ref, wf_ref, bf_ref, att_ref, ex_ref):
    s = g_ref[...] + jnp.dot(ea_ref[...], wf_ref[...],
                             preferred_element_type=jnp.float32) + bf_ref[...]
    m = jnp.where(s > 0, s, 0.2 * s)
    alpha = jnp.dot(m, att_ref[...], preferred_element_type=jnp.float32)
    ex_ref[...] = jnp.exp(alpha)


def _alpha(g, ea_pad, wf, bf, att_bd, block_rows):
    E = g.shape[0]
    KP = ea_pad.shape[1]
    grid = (pl.cdiv(E, block_rows),)
    return pl.pallas_call(
        _alpha_body,
        grid=grid,
        in_specs=[
            pl.BlockSpec((block_rows, HID), lambda i: (i, 0)),
            pl.BlockSpec((block_rows, KP), lambda i: (i, 0)),
            pl.BlockSpec((KP, HID), lambda i: (0, 0)),
            pl.BlockSpec((1, HID), lambda i: (0, 0)),
            pl.BlockSpec((HID, HEADS), lambda i: (0, 0)),
        ],
        out_specs=pl.BlockSpec((block_rows, HEADS), lambda i: (i, 0)),
        out_shape=jax.ShapeDtypeStruct((E, HEADS), jnp.float32),
    )(g, ea_pad, wf, bf[None, :], att_bd)


def _ln_body(h_ref, c_ref, bias_ref, g_ref, b_ref, o_ref):
    v = h_ref[...] + c_ref[...] + bias_ref[...]
    mu = jnp.mean(v, -1, keepdims=True)
    var = jnp.mean((v - mu) ** 2, -1, keepdims=True)
    o_ref[...] = (v - mu) / jnp.sqrt(var + 1e-5) * g_ref[...] + b_ref[...]


def _ln_res(h, c, bias, g, b, block_rows):
    R = h.shape[0]
    grid = (pl.cdiv(R, block_rows),)
    return pl.pallas_call(
        _ln_body,
        grid=grid,
        in_specs=[
            pl.BlockSpec((block_rows, HID), lambda i: (i, 0)),
            pl.BlockSpec((block_rows, HID), lambda i: (i, 0)),
            pl.BlockSpec((1, HID), lambda i: (0, 0)),
            pl.BlockSpec((1, HID), lambda i: (0, 0)),
            pl.BlockSpec((1, HID), lambda i: (0, 0)),
        ],
        out_specs=pl.BlockSpec((block_rows, HID), lambda i: (i, 0)),
        out_shape=jax.ShapeDtypeStruct((R, HID), jnp.float32),
    )(h, c, bias[None, :], g[None, :], b[None, :])


def _silu(v):
    return v * jax.nn.sigmoid(v)


def _final_body(s_ref, cnt_ref, wm1_ref, bm1_ref, wm2_ref, bm2_ref,
                w1_ref, b1_ref, w2_ref, b2_ref, w3_ref, b3_ref, o_ref):
    s = s_ref[...]
    cnt = jnp.maximum(cnt_ref[...], 1.0)
    mean = s / cnt
    gcat = jnp.concatenate([mean, s], axis=-1)
    z = _silu(jnp.dot(gcat, wm1_ref[...], preferred_element_type=jnp.float32)
              + bm1_ref[...])
    z = jnp.dot(z, wm2_ref[...], preferred_element_type=jnp.float32) + bm2_ref[...]
    outs = []
    for t in range(w1_ref.shape[0]):
        u = _silu(jnp.dot(z, w1_ref[t], preferred_element_type=jnp.float32)
                  + b1_ref[t][None, :])
        u = _silu(jnp.dot(u, w2_ref[t], preferred_element_type=jnp.float32)
                  + b2_ref[t][None, :])
        o = jnp.dot(u, w3_ref[t][:, None],
                    preferred_element_type=jnp.float32) + b3_ref[t][None, :]
        outs.append(o)
    o_ref[...] = jnp.concatenate(outs, axis=-1)


def _final(s, cnt, p, tasks):
    G = s.shape[0]
    w1 = jnp.stack([hd["W1"] for hd in p["heads"]])
    b1 = jnp.stack([hd["b1"] for hd in p["heads"]])
    w2 = jnp.stack([hd["W2"] for hd in p["heads"]])
    b2 = jnp.stack([hd["b2"] for hd in p["heads"]])
    w3 = jnp.stack([hd["W3"][:, 0] for hd in p["heads"]])
    b3 = jnp.stack([hd["b3"] for hd in p["heads"]])

    def full(*shape):
        return pl.BlockSpec(shape, lambda: tuple(0 for _ in shape))

    return pl.pallas_call(
        _final_body,
        in_specs=[
            full(G, HID), full(G, 1),
            full(2 * HID, HID), full(1, HID),
            full(HID, HID), full(1, HID),
            full(tasks, HID, HID), full(tasks, HID),
            full(tasks, HID, 64), full(tasks, 64),
            full(tasks, 64), full(tasks, 1),
        ],
        out_specs=full(G, tasks),
        out_shape=jax.ShapeDtypeStruct((G, tasks), jnp.float32),
    )(s, cnt[:, None], p["Wm1"], p["bm1"][None, :], p["Wm2"], p["bm2"][None, :],
      w1, b1, w2, b2, w3, b3)


def kernel(x, edge_index, edge_attr, batch, params):
    N = x.shape[0]
    E = edge_attr.shape[0]
    G = 1024
    TASKS = len(params["heads"])
    src = edge_index[0]
    dst = edge_index[1]

    xp = jnp.pad(x, ((0, 0), (0, 128 - x.shape[1])))
    w_ne = jnp.pad(params["W_ne"], ((0, 128 - x.shape[1]), (0, 0)))
    ea = jnp.pad(edge_attr, ((0, 0), (0, 16 - edge_attr.shape[1])))

    h = _mm_bias(xp, w_ne, params["b_ne"], 1000)

    for li in range(LAYERS):
        c = params["convs"][li]
        att = c["att"]
        att_bd = jnp.zeros((HID, HEADS), jnp.float32)
        for hh in range(HEADS):
            att_bd = att_bd.at[hh * CH:(hh + 1) * CH, hh].set(att[hh])
        wf = jnp.pad(params["W_ee"] @ c["We"], ((0, 6), (0, 0)))
        bf = params["b_ee"] @ c["We"]
        xl, xr = _mm2(h, c["Wl"], c["bl"], c["Wr"], c["br"], 1000)

        g = xl[src] + xr[dst]
        ex = _alpha(g, ea, wf, bf, att_bd, 3200)
        den = jax.ops.segment_sum(ex, dst, num_segments=N)
        a = ex / (den[dst] + 1e-16)
        msg = (xl[src].reshape(E, HEADS, CH) * a[..., None]).reshape(E, HID)
        num = jax.ops.segment_sum(msg, dst, num_segments=N)

        h = _ln_res(h, num, c["bias"], params["norms"][li]["g"],
                    params["norms"][li]["b"], 1000)

    s = jax.ops.segment_sum(h, batch, num_segments=G)
    cnt = jax.ops.segment_sum(jnp.ones((N,), jnp.float32), batch, num_segments=G)
    return _final(s, cnt, params, TASKS)


# speedup vs baseline: 3.4895x; 1.7937x over previous
import functools
import jax
import jax.numpy as jnp
from jax import lax
from jax.experimental import pallas as pl
from jax.experimental.pallas import tpu as pltpu
from jax.experimental.pallas import tpu_sc as plsc

HID = 128
HEADS = 4
CH = 32
LAYERS = 4
POOL_W = 256

NW = 32
CHUNK = 128


def _mm_bias_body(x_ref, w_ref, b_ref, o_ref):
    o_ref[...] = jnp.dot(x_ref[...], w_ref[...],
                         preferred_element_type=jnp.float32) + b_ref[...]


def _mm_bias(x, w, b, block_rows):
    R, K = x.shape
    M = w.shape[1]
    return pl.pallas_call(
        _mm_bias_body,
        grid=(pl.cdiv(R, block_rows),),
        in_specs=[
            pl.BlockSpec((block_rows, K), lambda i: (i, 0)),
            pl.BlockSpec((K, M), lambda i: (0, 0)),
            pl.BlockSpec((1, M), lambda i: (0, 0)),
        ],
        out_specs=pl.BlockSpec((block_rows, M), lambda i: (i, 0)),
        out_shape=jax.ShapeDtypeStruct((R, M), jnp.float32),
    )(x, w, b[None, :])


def _mm2_body(h_ref, wl_ref, bl_ref, wr_ref, br_ref, xl_ref, xr_ref):
    h = h_ref[...]
    xl_ref[...] = jnp.dot(h, wl_ref[...],
                          preferred_element_type=jnp.float32) + bl_ref[...]
    xr_ref[...] = jnp.dot(h, wr_ref[...],
                          preferred_element_type=jnp.float32) + br_ref[...]


def _mm2(h, wl, bl, wr, br, block_rows):
    R = h.shape[0]
    return pl.pallas_call(
        _mm2_body,
        grid=(pl.cdiv(R, block_rows),),
        in_specs=[
            pl.BlockSpec((block_rows, HID), lambda i: (i, 0)),
            pl.BlockSpec((HID, HID), lambda i: (0, 0)),
            pl.BlockSpec((1, HID), lambda i: (0, 0)),
            pl.BlockSpec((HID, HID), lambda i: (0, 0)),
            pl.BlockSpec((1, HID), lambda i: (0, 0)),
        ],
        out_specs=[
            pl.BlockSpec((block_rows, HID), lambda i: (i, 0)),
            pl.BlockSpec((block_rows, HID), lambda i: (i, 0)),
        ],
        out_shape=[
            jax.ShapeDtypeStruct((R, HID), jnp.float32),
            jax.ShapeDtypeStruct((R, HID), jnp.float32),
        ],
    )(h, wl, bl[None, :], wr, br[None, :])


def _alpha_body(e_real_ref, gl_ref, gr_ref, ea_ref, wf_ref, bf_ref, att_ref,
                sel_ref, ex_ref, msg_ref):
    i = pl.program_id(0)
    be = gl_ref.shape[0]
    gl = gl_ref[...]
    s = gl + gr_ref[...] + jnp.dot(ea_ref[...], wf_ref[...],
                                   preferred_element_type=jnp.float32) + bf_ref[...]
    m = jnp.where(s > 0, s, 0.2 * s)
    alpha = jnp.dot(m, att_ref[...], preferred_element_type=jnp.float32)
    rows = i * be + lax.broadcasted_iota(jnp.int32, (be, 1), 0)
    ev = jnp.where(rows < e_real_ref[0], jnp.exp(alpha), 0.0)
    ex_ref[...] = jnp.concatenate(
        [ev, jnp.zeros((be, HID - HEADS), jnp.float32)], axis=-1)
    msg_ref[...] = gl * jnp.dot(ev, sel_ref[...],
                                preferred_element_type=jnp.float32)


def _alpha(e_real, gl, gr, ea_pad, wf, bf, att_bd, sel4, block_rows):
    EP = gl.shape[0]
    KP = ea_pad.shape[1]
    return pl.pallas_call(
        _alpha_body,
        grid=(pl.cdiv(EP, block_rows),),
        in_specs=[
            pl.BlockSpec(memory_space=pltpu.SMEM),
            pl.BlockSpec((block_rows, HID), lambda i: (i, 0)),
            pl.BlockSpec((block_rows, HID), lambda i: (i, 0)),
            pl.BlockSpec((block_rows, KP), lambda i: (i, 0)),
            pl.BlockSpec((KP, HID), lambda i: (0, 0)),
            pl.BlockSpec((1, HID), lambda i: (0, 0)),
            pl.BlockSpec((HID, HEADS), lambda i: (0, 0)),
            pl.BlockSpec((HEADS, HID), lambda i: (0, 0)),
        ],
        out_specs=[pl.BlockSpec((block_rows, HID), lambda i: (i, 0)),
                   pl.BlockSpec((block_rows, HID), lambda i: (i, 0))],
        out_shape=[jax.ShapeDtypeStruct((EP, HID), jnp.float32),
                   jax.ShapeDtypeStruct((EP, HID), jnp.float32)],
    )(e_real, gl, gr, ea_pad, wf, bf[None, :], att_bd, sel4)


def _ln_body(h_ref, num_ref, d_ref, sel_ref, bias_ref, g_ref, b_ref, o_ref):
    den = d_ref[0]
    r = 1.0 / (den[:, :HEADS] + 1e-16)
    rex = jnp.dot(r, sel_ref[...], preferred_element_type=jnp.float32)
    v = h_ref[...] + num_ref[0] * rex + bias_ref[...]
    mu = jnp.mean(v, -1, keepdims=True)
    var = jnp.mean((v - mu) ** 2, -1, keepdims=True)
    o_ref[...] = (v - mu) / jnp.sqrt(var + 1e-5) * g_ref[...] + b_ref[...]


def _ln_res(h, num4, den4, sel4, bias, g, b, rn):
    br = rn // 16
    R = h.shape[0]
    return pl.pallas_call(
        _ln_body,
        grid=(pl.cdiv(R, br),),
        in_specs=[
            pl.BlockSpec((br, HID), lambda i: (i, 0)),
            pl.BlockSpec((1, br, HID), lambda i: (i // 16, i % 16, 0)),
            pl.BlockSpec((1, br, HID), lambda i: (i // 16, i % 16, 0)),
            pl.BlockSpec((HEADS, HID), lambda i: (0, 0)),
            pl.BlockSpec((1, HID), lambda i: (0, 0)),
            pl.BlockSpec((1, HID), lambda i: (0, 0)),
            pl.BlockSpec((1, HID), lambda i: (0, 0)),
        ],
        out_specs=pl.BlockSpec((br, HID), lambda i: (i, 0)),
        out_shape=jax.ShapeDtypeStruct((R, HID), jnp.float32),
    )(h, num4, den4, sel4, bias[None, :], g[None, :], b[None, :])


def _silu(v):
    return v * jax.nn.sigmoid(v)


def _final_body(s_ref, wm1_ref, bm1_ref, wm2_ref, bm2_ref,
                w1_ref, b1_ref, w2_ref, b2_ref, w3_ref, b3_ref, o_ref):
    G = o_ref.shape[0]
    acc = s_ref[0] + s_ref[1]
    s = acc[:G]
    cnt = jnp.maximum(acc[G:2 * G, :1], 1.0)
    mean = s / cnt
    gcat = jnp.concatenate([mean, s], axis=-1)
    z = _silu(jnp.dot(gcat, wm1_ref[...], preferred_element_type=jnp.float32)
              + bm1_ref[...])
    z = jnp.dot(z, wm2_ref[...], preferred_element_type=jnp.float32) + bm2_ref[...]
    outs = []
    for t in range(w1_ref.shape[0]):
        u = _silu(jnp.dot(z, w1_ref[t], preferred_element_type=jnp.float32)
                  + b1_ref[t][None, :])
        u = _silu(jnp.dot(u, w2_ref[t], preferred_element_type=jnp.float32)
                  + b2_ref[t][None, :])
        o = jnp.dot(u, w3_ref[t][:, None],
                    preferred_element_type=jnp.float32) + b3_ref[t][None, :]
        outs.append(o)
    o_ref[...] = jnp.concatenate(outs, axis=-1)


def _final(s2, p, tasks):
    G = (s2.shape[1] - 128) // 2
    w1 = jnp.stack([hd["W1"] for hd in p["heads"]])
    b1 = jnp.stack([hd["b1"] for hd in p["heads"]])
    w2 = jnp.stack([hd["W2"] for hd in p["heads"]])
    b2 = jnp.stack([hd["b2"] for hd in p["heads"]])
    w3 = jnp.stack([hd["W3"][:, 0] for hd in p["heads"]])
    b3 = jnp.stack([hd["b3"] for hd in p["heads"]])

    def full(*shape):
        return pl.BlockSpec(shape, lambda: tuple(0 for _ in shape))

    return pl.pallas_call(
        _final_body,
        in_specs=[
            full(2, 2 * G + 128, HID),
            full(2 * HID, HID), full(1, HID),
            full(HID, HID), full(1, HID),
            full(tasks, HID, HID), full(tasks, HID),
            full(tasks, HID, 64), full(tasks, 64),
            full(tasks, 64), full(tasks, 1),
        ],
        out_specs=full(G, tasks),
        out_shape=jax.ShapeDtypeStruct((G, tasks), jnp.float32),
    )(s2, p["Wm1"], p["bm1"][None, :], p["Wm2"], p["bm2"][None, :],
      w1, b1, w2, b2, w3, b3)


def _sc_mesh():
    return plsc.VectorSubcoreMesh(core_axis_name="c", subcore_axis_name="s")


def _sc_gather2(xl, xr, src_p, dst_p):
    EP = src_p.shape[0]
    epw = EP // NW
    nchunk = epw // CHUNK

    @functools.partial(
        pl.kernel, mesh=_sc_mesh(),
        out_type=[jax.ShapeDtypeStruct((EP, HID), jnp.float32),
                  jax.ShapeDtypeStruct((EP, HID), jnp.float32)],
        scratch_types=[
            pltpu.VMEM((CHUNK,), jnp.int32), pltpu.VMEM((CHUNK,), jnp.int32),
            pltpu.VMEM((CHUNK, HID), jnp.float32),
            pltpu.VMEM((CHUNK, HID), jnp.float32),
            pltpu.SemaphoreType.DMA, pltpu.SemaphoreType.DMA,
        ],
    )
    def k(xl_hbm, xr_hbm, src_hbm, dst_hbm, gl_hbm, gr_hbm,
          si, di, bl, br, s1, s2):
        wid = lax.axis_index("s") * 2 + lax.axis_index("c")
        base = wid * epw

        def body(t, carry):
            off = base + t * CHUNK
            pltpu.sync_copy(src_hbm.at[pl.ds(off, CHUNK)], si)
            pltpu.sync_copy(dst_hbm.at[pl.ds(off, CHUNK)], di)
            c1 = pltpu.async_copy(xl_hbm.at[si], bl, s1)
            c2 = pltpu.async_copy(xr_hbm.at[di], br, s2)
            c1.wait()
            c2.wait()
            pltpu.sync_copy(bl, gl_hbm.at[pl.ds(off, CHUNK)])
            pltpu.sync_copy(br, gr_hbm.at[pl.ds(off, CHUNK)])
            return carry

        lax.fori_loop(0, nchunk, body, 0)

    return k(xl, xr, src_p, dst_p)


def _zfill(buf, rows):
    z = jnp.zeros((16,), jnp.float32)

    def zb(t, carry):
        r = t // (HID // 16)
        cc = t % (HID // 16)
        buf[r, pl.ds(cc * 16, 16)] = z
        return carry

    lax.fori_loop(0, rows * (HID // 16), zb, 0)


def _sc_num(msg, dst2d, zrows, n_pad, rn, rtrash):
    EP = dst2d.shape[0] * CHUNK
    ept = EP // 16
    nchunk = ept // CHUNK
    rows_per_tile = rtrash // 16

    @functools.partial(
        pl.kernel, mesh=_sc_mesh(),
        out_type=jax.ShapeDtypeStruct((4, rtrash, HID), jnp.float32),
        scratch_types=[
            pltpu.VMEM((CHUNK,), jnp.int32),
            pltpu.VMEM((CHUNK,), jnp.int32),
            pltpu.VMEM((CHUNK, HID), jnp.float32),
            pltpu.VMEM_SHARED((rtrash, HID), jnp.float32),
        ],
    )
    def k(msg_hbm, dst_hbm, z_hbm, num_hbm, di, di2, mb, acc):
        cid = lax.axis_index("c")
        sid = lax.axis_index("s")
        base = sid * ept
        basec = sid * (ept // CHUNK)
        row0 = sid * rows_per_tile
        for rep in range(2):
            rng = cid * 2 + rep
            lo = rng * rn
            pltpu.sync_copy(z_hbm, acc.at[pl.ds(row0, rows_per_tile)])
            plsc.subcore_barrier()

            def body(t, carry):
                off = base + t * CHUNK
                pltpu.sync_copy(dst_hbm.at[basec + t], di)
                pltpu.sync_copy(msg_hbm.at[pl.ds(off, CHUNK)], mb)
                for j in range(CHUNK // 16):
                    d = di[pl.ds(j * 16, 16)]
                    dl = d - lo
                    ok = (dl >= 0) & (dl < rn)
                    di2[pl.ds(j * 16, 16)] = jnp.where(ok, dl, rn)
                pltpu.sync_copy(mb, acc.at[di2], add=True)
                return carry

            lax.fori_loop(0, nchunk, body, 0)
            plsc.subcore_barrier()
            pltpu.sync_copy(acc.at[pl.ds(row0, rows_per_tile)],
                            num_hbm.at[rng, pl.ds(row0, rows_per_tile)])
            plsc.subcore_barrier()

    return k(msg, dst2d, zrows)


def _sc_pool(hp, batch_p, g_seg):
    NP = batch_p.shape[0]
    npw = NP // NW
    nchunk = npw // CHUNK
    nrows = 2 * g_seg + 128
    rows_per_tile = nrows // 16

    @functools.partial(
        pl.kernel, mesh=_sc_mesh(),
        out_type=jax.ShapeDtypeStruct((2, nrows, HID), jnp.float32),
        scratch_types=[
            pltpu.VMEM((CHUNK,), jnp.int32),
            pltpu.VMEM((CHUNK,), jnp.int32),
            pltpu.VMEM((CHUNK, HID), jnp.float32),
            pltpu.VMEM((CHUNK, HID), jnp.float32),
            pltpu.VMEM_SHARED((nrows, HID), jnp.float32),
        ],
    )
    def k(hp_hbm, b_hbm, s_hbm, bi, bi2, hb, ob, acc):
        cid = lax.axis_index("c")
        sid = lax.axis_index("s")
        wid = sid * 2 + cid
        base = wid * npw
        row0 = sid * rows_per_tile
        _zfill(ob, CHUNK)
        one = jnp.full((16,), 1.0, jnp.float32)
        zero = jnp.zeros((16,), jnp.float32)
        idx16 = lax.iota(jnp.int32, 16)

        def fill1(r, carry):
            ob[r, pl.ds(0, 16)] = jnp.where(idx16 == 0, one, zero)
            return carry

        lax.fori_loop(0, CHUNK, fill1, 0)
        _zfill(hb, CHUNK)
        pltpu.sync_copy(hb.at[pl.ds(0, CHUNK)], acc.at[pl.ds(row0, CHUNK)])
        pltpu.sync_copy(hb.at[pl.ds(0, 8)], acc.at[pl.ds(row0 + CHUNK, 8)])
        plsc.subcore_barrier()

        def body(t, carry):
            off = base + t * CHUNK
            pltpu.sync_copy(b_hbm.at[pl.ds(off, CHUNK)], bi)
            pltpu.sync_copy(hp_hbm.at[pl.ds(off, CHUNK)], hb)
            pltpu.sync_copy(hb, acc.at[bi], add=True)
            for j in range(CHUNK // 16):
                b16 = bi[pl.ds(j * 16, 16)]
                bi2[pl.ds(j * 16, 16)] = jnp.where(
                    b16 < g_seg, b16 + g_seg, 2 * g_seg)
            pltpu.sync_copy(ob, acc.at[bi2], add=True)
            return carry

        lax.fori_loop(0, nchunk, body, 0)
        plsc.subcore_barrier()
        pltpu.sync_copy(acc.at[pl.ds(row0, rows_per_tile)],
                        s_hbm.at[cid, pl.ds(row0, rows_per_tile)])

    return k(hp, batch_p)


def kernel(x, edge_index, edge_attr, batch, params):
    N = x.shape[0]
    E = edge_attr.shape[0]
    G = 1024
    TASKS = len(params["heads"])

    EP = ((E + NW * CHUNK - 1) // (NW * CHUNK)) * (NW * CHUNK)
    NPAD = ((N + 255) // 256) * 256
    RN = NPAD // 4
    RTRASH = RN + 128
    NPOOL = ((N + NW * CHUNK - 1) // (NW * CHUNK)) * (NW * CHUNK)

    src_p = jnp.pad(edge_index[0], (0, EP - E))
    dst_p = jnp.pad(edge_index[1], (0, EP - E))
    dst2d = dst_p.reshape(EP // CHUNK, CHUNK)
    ea = jnp.pad(edge_attr, ((0, EP - E), (0, 16 - edge_attr.shape[1])))
    e_real = jnp.array([E], jnp.int32)

    xp = jnp.pad(x, ((0, 0), (0, 128 - x.shape[1])))
    w_ne = jnp.pad(params["W_ne"], ((0, 128 - x.shape[1]), (0, 0)))

    sel4 = jnp.zeros((HEADS, HID), jnp.float32)
    for hh in range(HEADS):
        sel4 = sel4.at[hh, hh * CH:(hh + 1) * CH].set(1.0)

    zrows = jnp.zeros((RTRASH // 16, HID), jnp.float32)

    h = _mm_bias(xp, w_ne, params["b_ne"], 1000)

    for li in range(LAYERS):
        c = params["convs"][li]
        att = c["att"]
        att_bd = jnp.zeros((HID, HEADS), jnp.float32)
        for hh in range(HEADS):
            att_bd = att_bd.at[hh * CH:(hh + 1) * CH, hh].set(att[hh])
        wf = jnp.pad(params["W_ee"] @ c["We"], ((0, 6), (0, 0)))
        bf = params["b_ee"] @ c["We"]

        xl, xr = _mm2(h, c["Wl"], c["bl"], c["Wr"], c["br"], 1000)
        gl, gr = _sc_gather2(xl, xr, src_p, dst_p)
        ex, msg = _alpha(e_real, gl, gr, ea, wf, bf, att_bd, sel4, 3136)
        den4 = _sc_num(ex, dst2d, zrows, NPAD, RN, RTRASH)
        num4 = _sc_num(msg, dst2d, zrows, NPAD, RN, RTRASH)
        h = _ln_res(h, num4, den4, sel4, c["bias"],
                    params["norms"][li]["g"], params["norms"][li]["b"], RN)

    hp = jnp.pad(h, ((0, NPOOL - N), (0, 0)))
    batch_p = jnp.pad(batch, (0, NPOOL - N), constant_values=2 * G)
    s2 = _sc_pool(hp, batch_p, G)
    return _final(s2, params, TASKS)


# speedup vs baseline: 4.2208x; 1.2096x over previous
import functools
import jax
import jax.numpy as jnp
from jax import lax
from jax.experimental import pallas as pl
from jax.experimental.pallas import tpu as pltpu
from jax.experimental.pallas import tpu_sc as plsc

HID = 128
HEADS = 4
CH = 32
LAYERS = 4
POOL_W = 256

NW = 32
CHUNK = 128


def _mm_bias_body(x_ref, w_ref, b_ref, o_ref):
    o_ref[...] = jnp.dot(x_ref[...], w_ref[...],
                         preferred_element_type=jnp.float32) + b_ref[...]


def _mm_bias(x, w, b, block_rows):
    R, K = x.shape
    M = w.shape[1]
    return pl.pallas_call(
        _mm_bias_body,
        grid=(pl.cdiv(R, block_rows),),
        in_specs=[
            pl.BlockSpec((block_rows, K), lambda i: (i, 0)),
            pl.BlockSpec((K, M), lambda i: (0, 0)),
            pl.BlockSpec((1, M), lambda i: (0, 0)),
        ],
        out_specs=pl.BlockSpec((block_rows, M), lambda i: (i, 0)),
        out_shape=jax.ShapeDtypeStruct((R, M), jnp.float32),
    )(x, w, b[None, :])


def _mm2_body(h_ref, wl_ref, bl_ref, wr_ref, br_ref, xl_ref, xr_ref):
    h = h_ref[...]
    xl_ref[...] = jnp.dot(h, wl_ref[...],
                          preferred_element_type=jnp.float32) + bl_ref[...]
    xr_ref[...] = jnp.dot(h, wr_ref[...],
                          preferred_element_type=jnp.float32) + br_ref[...]


def _mm2(h, wl, bl, wr, br, block_rows):
    R = h.shape[0]
    return pl.pallas_call(
        _mm2_body,
        grid=(pl.cdiv(R, block_rows),),
        in_specs=[
            pl.BlockSpec((block_rows, HID), lambda i: (i, 0)),
            pl.BlockSpec((HID, HID), lambda i: (0, 0)),
            pl.BlockSpec((1, HID), lambda i: (0, 0)),
            pl.BlockSpec((HID, HID), lambda i: (0, 0)),
            pl.BlockSpec((1, HID), lambda i: (0, 0)),
        ],
        out_specs=[
            pl.BlockSpec((block_rows, HID), lambda i: (i, 0)),
            pl.BlockSpec((block_rows, HID), lambda i: (i, 0)),
        ],
        out_shape=[
            jax.ShapeDtypeStruct((R, HID), jnp.float32),
            jax.ShapeDtypeStruct((R, HID), jnp.float32),
        ],
    )(h, wl, bl[None, :], wr, br[None, :])


def _alpha_body(e_real_ref, gl_ref, gr_ref, ea_ref, wf_ref, bf_ref, att_ref,
                sel_ref, ex_ref, msg_ref):
    i = pl.program_id(0)
    be = gl_ref.shape[0]
    gl = gl_ref[...]
    s = gl + gr_ref[...] + jnp.dot(ea_ref[...], wf_ref[...],
                                   preferred_element_type=jnp.float32) + bf_ref[...]
    m = jnp.where(s > 0, s, 0.2 * s)
    alpha = jnp.dot(m, att_ref[...], preferred_element_type=jnp.float32)
    rows = i * be + lax.broadcasted_iota(jnp.int32, (be, 1), 0)
    ev = jnp.where(rows < e_real_ref[0], jnp.exp(alpha), 0.0)
    ex_ref[...] = jnp.concatenate(
        [ev, jnp.zeros((be, HID - HEADS), jnp.float32)], axis=-1)
    msg_ref[...] = gl * jnp.dot(ev, sel_ref[...],
                                preferred_element_type=jnp.float32)


def _alpha(e_real, gl, gr, ea_pad, wf, bf, att_bd, sel4, block_rows):
    EP = gl.shape[0]
    KP = ea_pad.shape[1]
    return pl.pallas_call(
        _alpha_body,
        grid=(pl.cdiv(EP, block_rows),),
        in_specs=[
            pl.BlockSpec(memory_space=pltpu.SMEM),
            pl.BlockSpec((block_rows, HID), lambda i: (i, 0)),
            pl.BlockSpec((block_rows, HID), lambda i: (i, 0)),
            pl.BlockSpec((block_rows, KP), lambda i: (i, 0)),
            pl.BlockSpec((KP, HID), lambda i: (0, 0)),
            pl.BlockSpec((1, HID), lambda i: (0, 0)),
            pl.BlockSpec((HID, HEADS), lambda i: (0, 0)),
            pl.BlockSpec((HEADS, HID), lambda i: (0, 0)),
        ],
        out_specs=[pl.BlockSpec((block_rows, HID), lambda i: (i, 0)),
                   pl.BlockSpec((block_rows, HID), lambda i: (i, 0))],
        out_shape=[jax.ShapeDtypeStruct((EP, HID), jnp.float32),
                   jax.ShapeDtypeStruct((EP, HID), jnp.float32)],
    )(e_real, gl, gr, ea_pad, wf, bf[None, :], att_bd, sel4)


def _ln_body(h_ref, num_ref, d_ref, sel_ref, bias_ref, g_ref, b_ref, o_ref):
    den = d_ref[0]
    r = 1.0 / (den[:, :HEADS] + 1e-16)
    rex = jnp.dot(r, sel_ref[...], preferred_element_type=jnp.float32)
    v = h_ref[...] + num_ref[0] * rex + bias_ref[...]
    mu = jnp.mean(v, -1, keepdims=True)
    var = jnp.mean((v - mu) ** 2, -1, keepdims=True)
    o_ref[...] = (v - mu) / jnp.sqrt(var + 1e-5) * g_ref[...] + b_ref[...]


def _ln_res(h, num4, den4, sel4, bias, g, b, rn):
    br = rn // 16
    R = h.shape[0]
    return pl.pallas_call(
        _ln_body,
        grid=(pl.cdiv(R, br),),
        in_specs=[
            pl.BlockSpec((br, HID), lambda i: (i, 0)),
            pl.BlockSpec((1, br, HID), lambda i: (i // 16, i % 16, 0)),
            pl.BlockSpec((1, br, HID), lambda i: (i // 16, i % 16, 0)),
            pl.BlockSpec((HEADS, HID), lambda i: (0, 0)),
            pl.BlockSpec((1, HID), lambda i: (0, 0)),
            pl.BlockSpec((1, HID), lambda i: (0, 0)),
            pl.BlockSpec((1, HID), lambda i: (0, 0)),
        ],
        out_specs=pl.BlockSpec((br, HID), lambda i: (i, 0)),
        out_shape=jax.ShapeDtypeStruct((R, HID), jnp.float32),
    )(h, num4, den4, sel4, bias[None, :], g[None, :], b[None, :])


def _silu(v):
    return v * jax.nn.sigmoid(v)


def _final_body(s_ref, wm1_ref, bm1_ref, wm2_ref, bm2_ref,
                w1_ref, b1_ref, w2_ref, b2_ref, w3_ref, b3_ref, o_ref):
    G = o_ref.shape[0]
    acc = s_ref[0] + s_ref[1]
    s = acc[:G]
    cnt = jnp.maximum(acc[G:2 * G, :1], 1.0)
    mean = s / cnt
    gcat = jnp.concatenate([mean, s], axis=-1)
    z = _silu(jnp.dot(gcat, wm1_ref[...], preferred_element_type=jnp.float32)
              + bm1_ref[...])
    z = jnp.dot(z, wm2_ref[...], preferred_element_type=jnp.float32) + bm2_ref[...]
    outs = []
    for t in range(w1_ref.shape[0]):
        u = _silu(jnp.dot(z, w1_ref[t], preferred_element_type=jnp.float32)
                  + b1_ref[t][None, :])
        u = _silu(jnp.dot(u, w2_ref[t], preferred_element_type=jnp.float32)
                  + b2_ref[t][None, :])
        o = jnp.dot(u, w3_ref[t][:, None],
                    preferred_element_type=jnp.float32) + b3_ref[t][None, :]
        outs.append(o)
    o_ref[...] = jnp.concatenate(outs, axis=-1)


def _final(s2, p, tasks):
    G = (s2.shape[1] - 128) // 2
    w1 = jnp.stack([hd["W1"] for hd in p["heads"]])
    b1 = jnp.stack([hd["b1"] for hd in p["heads"]])
    w2 = jnp.stack([hd["W2"] for hd in p["heads"]])
    b2 = jnp.stack([hd["b2"] for hd in p["heads"]])
    w3 = jnp.stack([hd["W3"][:, 0] for hd in p["heads"]])
    b3 = jnp.stack([hd["b3"] for hd in p["heads"]])

    def full(*shape):
        return pl.BlockSpec(shape, lambda: tuple(0 for _ in shape))

    return pl.pallas_call(
        _final_body,
        in_specs=[
            full(2, 2 * G + 128, HID),
            full(2 * HID, HID), full(1, HID),
            full(HID, HID), full(1, HID),
            full(tasks, HID, HID), full(tasks, HID),
            full(tasks, HID, 64), full(tasks, 64),
            full(tasks, 64), full(tasks, 1),
        ],
        out_specs=full(G, tasks),
        out_shape=jax.ShapeDtypeStruct((G, tasks), jnp.float32),
    )(s2, p["Wm1"], p["bm1"][None, :], p["Wm2"], p["bm2"][None, :],
      w1, b1, w2, b2, w3, b3)


def _sc_mesh():
    return plsc.VectorSubcoreMesh(core_axis_name="c", subcore_axis_name="s")


def _sc_gather2(xl, xr, src_p, dst_p):
    EP = src_p.shape[0]
    epw = EP // NW
    nchunk = epw // CHUNK

    @functools.partial(
        pl.kernel, mesh=_sc_mesh(),
        out_type=[jax.ShapeDtypeStruct((EP, HID), jnp.float32),
                  jax.ShapeDtypeStruct((EP, HID), jnp.float32)],
        scratch_types=[
            pltpu.VMEM((CHUNK,), jnp.int32), pltpu.VMEM((CHUNK,), jnp.int32),
            pltpu.VMEM((CHUNK,), jnp.int32), pltpu.VMEM((CHUNK,), jnp.int32),
            pltpu.VMEM((CHUNK, HID), jnp.float32),
            pltpu.VMEM((CHUNK, HID), jnp.float32),
            pltpu.VMEM((CHUNK, HID), jnp.float32),
            pltpu.VMEM((CHUNK, HID), jnp.float32),
            pltpu.SemaphoreType.DMA, pltpu.SemaphoreType.DMA,
            pltpu.SemaphoreType.DMA, pltpu.SemaphoreType.DMA,
        ],
    )
    def k(xl_hbm, xr_hbm, src_hbm, dst_hbm, gl_hbm, gr_hbm,
          si0, di0, si1, di1, bl0, br0, bl1, br1, sl0, sr0, sl1, sr1):
        wid = lax.axis_index("s") * 2 + lax.axis_index("c")
        base = wid * epw
        lastoff = base + (nchunk - 1) * CHUNK

        def start(off, si, di, bl, br, sl, sr):
            pltpu.sync_copy(src_hbm.at[pl.ds(off, CHUNK)], si)
            pltpu.sync_copy(dst_hbm.at[pl.ds(off, CHUNK)], di)
            pltpu.async_copy(xl_hbm.at[si], bl, sl)
            pltpu.async_copy(xr_hbm.at[di], br, sr)

        def finish(off, bl, br, sl, sr):
            pltpu.make_async_copy(xl_hbm.at[pl.ds(0, CHUNK)], bl, sl).wait()
            pltpu.make_async_copy(xr_hbm.at[pl.ds(0, CHUNK)], br, sr).wait()
            pltpu.sync_copy(bl, gl_hbm.at[pl.ds(off, CHUNK)])
            pltpu.sync_copy(br, gr_hbm.at[pl.ds(off, CHUNK)])

        start(base, si0, di0, bl0, br0, sl0, sr0)

        def body(t, carry):
            off = base + 2 * t * CHUNK
            off1 = off + CHUNK
            offn = jnp.minimum(off + 2 * CHUNK, lastoff)
            start(off1, si1, di1, bl1, br1, sl1, sr1)
            finish(off, bl0, br0, sl0, sr0)
            start(offn, si0, di0, bl0, br0, sl0, sr0)
            finish(off1, bl1, br1, sl1, sr1)
            return carry

        lax.fori_loop(0, nchunk // 2, body, 0)
        pltpu.make_async_copy(xl_hbm.at[pl.ds(0, CHUNK)], bl0, sl0).wait()
        pltpu.make_async_copy(xr_hbm.at[pl.ds(0, CHUNK)], br0, sr0).wait()

    return k(xl, xr, src_p, dst_p)


def _zfill(buf, rows):
    z = jnp.zeros((16,), jnp.float32)

    def zb(t, carry):
        r = t // (HID // 16)
        cc = t % (HID // 16)
        buf[r, pl.ds(cc * 16, 16)] = z
        return carry

    lax.fori_loop(0, rows * (HID // 16), zb, 0)


def _sc_num(msg, dst2d, zrows, rn, rtrash):
    CK = dst2d.shape[1]
    EP = dst2d.shape[0] * CK
    ept = EP // 16
    nchunk = ept // CK
    rows_per_tile = rtrash // 16

    @functools.partial(
        pl.kernel, mesh=_sc_mesh(),
        out_type=jax.ShapeDtypeStruct((4, rtrash, HID), jnp.float32),
        scratch_types=[
            pltpu.VMEM((CK,), jnp.int32), pltpu.VMEM((CK,), jnp.int32),
            pltpu.VMEM((CK,), jnp.int32), pltpu.VMEM((CK,), jnp.int32),
            pltpu.VMEM((CK, HID), jnp.float32),
            pltpu.VMEM((CK, HID), jnp.float32),
            pltpu.SemaphoreType.DMA, pltpu.SemaphoreType.DMA,
            pltpu.VMEM_SHARED((rtrash, HID), jnp.float32),
        ],
    )
    def k(msg_hbm, dst_hbm, z_hbm, num_hbm,
          di0, di1, dl0, dl1, mb0, mb1, s0, s1, acc):
        cid = lax.axis_index("c")
        sid = lax.axis_index("s")
        basec = sid * nchunk
        row0 = sid * rows_per_tile
        last = basec + nchunk - 1

        def localize(di, dl, lo):
            for j in range(CK // 16):
                d = di[pl.ds(j * 16, 16)]
                dd = d - lo
                ok = (dd >= 0) & (dd < rn)
                dl[pl.ds(j * 16, 16)] = jnp.where(ok, dd, rn)

        for rep in range(2):
            rng = cid * 2 + rep
            lo = rng * rn
            pltpu.sync_copy(z_hbm, acc.at[pl.ds(row0, rows_per_tile)])
            plsc.subcore_barrier()
            pltpu.sync_copy(dst_hbm.at[basec], di0)
            g0 = pltpu.async_copy(
                msg_hbm.at[pl.ds(basec * CK, CK)], mb0, s0)

            def body(t, carry):
                c = basec + 2 * t
                c1 = c + 1
                cn = jnp.minimum(c + 2, last)
                pltpu.sync_copy(dst_hbm.at[c1], di1)
                g1 = pltpu.async_copy(
                    msg_hbm.at[pl.ds(c1 * CK, CK)], mb1, s1)
                pltpu.make_async_copy(
                    msg_hbm.at[pl.ds(c * CK, CK)], mb0, s0).wait()
                localize(di0, dl0, lo)
                pltpu.sync_copy(mb0, acc.at[dl0], add=True)
                pltpu.sync_copy(dst_hbm.at[cn], di0)
                pltpu.async_copy(
                    msg_hbm.at[pl.ds(cn * CK, CK)], mb0, s0)
                pltpu.make_async_copy(
                    msg_hbm.at[pl.ds(c1 * CK, CK)], mb1, s1).wait()
                localize(di1, dl1, lo)
                pltpu.sync_copy(mb1, acc.at[dl1], add=True)
                return carry

            lax.fori_loop(0, nchunk // 2, body, 0)
            pltpu.make_async_copy(
                msg_hbm.at[pl.ds(last * CK, CK)], mb0, s0).wait()
            plsc.subcore_barrier()
            pltpu.sync_copy(acc.at[pl.ds(row0, rows_per_tile)],
                            num_hbm.at[rng, pl.ds(row0, rows_per_tile)])
            plsc.subcore_barrier()

    return k(msg, dst2d, zrows)


def _sc_pool(hp, batch_p, g_seg):
    NP = batch_p.shape[0]
    npw = NP // NW
    nchunk = npw // CHUNK
    nrows = 2 * g_seg + 128
    rows_per_tile = nrows // 16

    @functools.partial(
        pl.kernel, mesh=_sc_mesh(),
        out_type=jax.ShapeDtypeStruct((2, nrows, HID), jnp.float32),
        scratch_types=[
            pltpu.VMEM((CHUNK,), jnp.int32),
            pltpu.VMEM((CHUNK,), jnp.int32),
            pltpu.VMEM((CHUNK, HID), jnp.float32),
            pltpu.VMEM((CHUNK, HID), jnp.float32),
            pltpu.VMEM_SHARED((nrows, HID), jnp.float32),
        ],
    )
    def k(hp_hbm, b_hbm, s_hbm, bi, bi2, hb, ob, acc):
        cid = lax.axis_index("c")
        sid = lax.axis_index("s")
        wid = sid * 2 + cid
        base = wid * npw
        row0 = sid * rows_per_tile
        _zfill(ob, CHUNK)
        one = jnp.full((16,), 1.0, jnp.float32)
        zero = jnp.zeros((16,), jnp.float32)
        idx16 = lax.iota(jnp.int32, 16)

        def fill1(r, carry):
            ob[r, pl.ds(0, 16)] = jnp.where(idx16 == 0, one, zero)
            return carry

        lax.fori_loop(0, CHUNK, fill1, 0)
        _zfill(hb, CHUNK)
        pltpu.sync_copy(hb.at[pl.ds(0, CHUNK)], acc.at[pl.ds(row0, CHUNK)])
        pltpu.sync_copy(hb.at[pl.ds(0, 8)], acc.at[pl.ds(row0 + CHUNK, 8)])
        plsc.subcore_barrier()

        def body(t, carry):
            off = base + t * CHUNK
            pltpu.sync_copy(b_hbm.at[pl.ds(off, CHUNK)], bi)
            pltpu.sync_copy(hp_hbm.at[pl.ds(off, CHUNK)], hb)
            pltpu.sync_copy(hb, acc.at[bi], add=True)
            for j in range(CHUNK // 16):
                b16 = bi[pl.ds(j * 16, 16)]
                bi2[pl.ds(j * 16, 16)] = jnp.where(
                    b16 < g_seg, b16 + g_seg, 2 * g_seg)
            pltpu.sync_copy(ob, acc.at[bi2], add=True)
            return carry

        lax.fori_loop(0, nchunk, body, 0)
        plsc.subcore_barrier()
        pltpu.sync_copy(acc.at[pl.ds(row0, rows_per_tile)],
                        s_hbm.at[cid, pl.ds(row0, rows_per_tile)])

    return k(hp, batch_p)


def kernel(x, edge_index, edge_attr, batch, params):
    N = x.shape[0]
    E = edge_attr.shape[0]
    G = 1024
    TASKS = len(params["heads"])

    EP = ((E + NW * CHUNK - 1) // (NW * CHUNK)) * (NW * CHUNK)
    NPAD = ((N + 255) // 256) * 256
    RN = NPAD // 4
    RTRASH = RN + 128
    NPOOL = ((N + NW * CHUNK - 1) // (NW * CHUNK)) * (NW * CHUNK)

    src_p = jnp.pad(edge_index[0], (0, EP - E))
    dst_p = jnp.pad(edge_index[1], (0, EP - E))
    dst2d = dst_p.reshape(EP // 112, 112)
    ea = jnp.pad(edge_attr, ((0, EP - E), (0, 16 - edge_attr.shape[1])))
    e_real = jnp.array([E], jnp.int32)

    xp = jnp.pad(x, ((0, 0), (0, 128 - x.shape[1])))
    w_ne = jnp.pad(params["W_ne"], ((0, 128 - x.shape[1]), (0, 0)))

    sel4 = jnp.zeros((HEADS, HID), jnp.float32)
    for hh in range(HEADS):
        sel4 = sel4.at[hh, hh * CH:(hh + 1) * CH].set(1.0)

    zrows = jnp.zeros((RTRASH // 16, HID), jnp.float32)

    h = _mm_bias(xp, w_ne, params["b_ne"], 1000)

    for li in range(LAYERS):
        c = params["convs"][li]
        att = c["att"]
        att_bd = jnp.zeros((HID, HEADS), jnp.float32)
        for hh in range(HEADS):
            att_bd = att_bd.at[hh * CH:(hh + 1) * CH, hh].set(att[hh])
        wf = jnp.pad(params["W_ee"] @ c["We"], ((0, 6), (0, 0)))
        bf = params["b_ee"] @ c["We"]

        xl, xr = _mm2(h, c["Wl"], c["bl"], c["Wr"], c["br"], 1000)
        gl, gr = _sc_gather2(xl, xr, src_p, dst_p)
        ex, msg = _alpha(e_real, gl, gr, ea, wf, bf, att_bd, sel4, 3136)
        den4 = _sc_num(ex, dst2d, zrows, RN, RTRASH)
        num4 = _sc_num(msg, dst2d, zrows, RN, RTRASH)
        h = _ln_res(h, num4, den4, sel4, c["bias"],
                    params["norms"][li]["g"], params["norms"][li]["b"], RN)

    hp = jnp.pad(h, ((0, NPOOL - N), (0, 0)))
    batch_p = jnp.pad(batch, (0, NPOOL - N), constant_values=2 * G)
    s2 = _sc_pool(hp, batch_p, G)
    return _final(s2, params, TASKS)
